# baseline (speedup 1.0000x reference)
import functools

import jax
import jax.numpy as jnp
from jax import lax
from jax.experimental import pallas as pl
from jax.experimental.pallas import tpu as pltpu
from jax.experimental.pallas import tpu_sc as plsc

N = 10000
E = 320000
H = 128
G = 64

NC = 2
NS = 16
NW = NC * NS
K = 64
NCH = 160
NST = 4
HC = NCH // NST
NB = 4
EPAD = NW * NCH * K
ND = 16
NA = N + ND
RSTEP = 624
RLEN = 640

R = 1000
GRID = N // R


def _agg_body(h_hbm, srcs_hbm, dsts_hbm, zeros_hbm, out_hbm,
              allis, allid, rows, sems, ssems, acc):
    c = lax.axis_index("c")
    s = lax.axis_index("s")
    wid = s * NC + c
    pltpu.sync_copy(zeros_hbm, acc.at[pl.ds(s * RSTEP, RLEN)])
    plsc.subcore_barrier()

    def half(hf, carry):
        pltpu.sync_copy(srcs_hbm.at[wid, pl.ds(hf * HC, HC)], allis)
        pltpu.sync_copy(dsts_hbm.at[wid, pl.ds(hf * HC, HC)], allid)
        for k in range(NB - 1):
            pltpu.async_copy(h_hbm.at[allis.at[k]], rows.at[k], sems.at[k])

        def body(i, carry2):
            base = NB * i
            for k in range(NB):
                j = base + k
                km1 = (k - 1) % NB
                pltpu.make_async_copy(h_hbm.at[allis.at[j]], rows.at[k],
                                      sems.at[k]).wait()
                pltpu.async_copy(rows.at[k], acc.at[allid.at[j]], ssems.at[k],
                                 add=True)
                jn = j + NB - 1

                @pl.when(jn < HC)
                def _():
                    @pl.when(j >= 1)
                    def _():
                        pltpu.make_async_copy(
                            rows.at[km1], acc.at[allid.at[j - 1]],
                            ssems.at[km1]).wait()

                    pltpu.async_copy(h_hbm.at[allis.at[jn]], rows.at[km1],
                                     sems.at[km1])
            return carry2

        lax.fori_loop(0, HC // NB, body, 0)
        for k in range(NB):
            pltpu.make_async_copy(rows.at[k], acc.at[allid.at[HC - NB + k]],
                                  ssems.at[k]).wait()
        return carry

    lax.fori_loop(0, NST, half, 0)
    plsc.subcore_barrier()
    pltpu.sync_copy(acc.at[pl.ds(s * RSTEP, RLEN)],
                    out_hbm.at[c, pl.ds(s * RSTEP, RLEN)])


@jax.jit
def _edge_agg(h, srcs, dsts, zeros):
    mesh = plsc.VectorSubcoreMesh(core_axis_name="c", subcore_axis_name="s")
    return pl.kernel(
        _agg_body,
        out_type=jax.ShapeDtypeStruct((NC, N, H), jnp.float32),
        mesh=mesh,
        scratch_types=[
            pltpu.VMEM((HC, K), jnp.int32),
            pltpu.VMEM((HC, K), jnp.int32),
            pltpu.VMEM((NB, K, H), jnp.float32),
            pltpu.SemaphoreType.DMA((NB,)),
            pltpu.SemaphoreType.DMA((NB,)),
            pltpu.VMEM_SHARED((NA, H), jnp.float32),
        ],
    )(h, srcs, dsts, zeros)


def _dense1_body(eps_ref, x_ref, p0_ref, p1_ref, w_ref, b_ref, o_ref):
    t = eps_ref[0, 0] * x_ref[...] + p0_ref[...] + p1_ref[...]
    acc = jnp.dot(t, w_ref[...], preferred_element_type=jnp.float32)
    o_ref[...] = jnp.maximum(acc + b_ref[...], 0.0)


@jax.jit
def _dense1(eps_s, x, p0, p1, w, b):
    return pl.pallas_call(
        _dense1_body,
        grid=(GRID,),
        in_specs=[
            pl.BlockSpec(memory_space=pltpu.MemorySpace.SMEM),
            pl.BlockSpec((R, H), lambda i: (i, 0)),
            pl.BlockSpec((R, H), lambda i: (i, 0)),
            pl.BlockSpec((R, H), lambda i: (i, 0)),
            pl.BlockSpec((H, H), lambda i: (0, 0)),
            pl.BlockSpec((1, H), lambda i: (0, 0)),
        ],
        out_specs=pl.BlockSpec((R, H), lambda i: (i, 0)),
        out_shape=jax.ShapeDtypeStruct((N, H), jnp.float32),
    )(eps_s, x, p0, p1, w, b)


def _dense2_body(eps_ref, h_ref, p0_ref, p1_ref, w_ref, b_ref, batch_ref,
                 wf_ref, bf_ref, o_ref, sums, counts):
    i = pl.program_id(0)
    t = eps_ref[0, 0] * h_ref[...] + p0_ref[...] + p1_ref[...]
    h2 = jnp.dot(t, w_ref[...], preferred_element_type=jnp.float32)
    h2 = jnp.maximum(h2 + b_ref[...], 0.0)
    ids = batch_ref[0, 0, :]
    gid = lax.broadcasted_iota(jnp.int32, (G, R), 0)
    mask = (ids[None, :] == gid).astype(jnp.float32)
    psum = jnp.dot(mask, h2, preferred_element_type=jnp.float32)
    pcnt = jnp.broadcast_to(jnp.sum(mask, axis=1, keepdims=True), (G, H))

    @pl.when(i == 0)
    def _():
        sums[...] = psum
        counts[...] = pcnt

    @pl.when(i > 0)
    def _():
        sums[...] += psum
        counts[...] += pcnt

    @pl.when(i == pl.num_programs(0) - 1)
    def _():
        pooled = sums[...] / jnp.maximum(counts[...], 1.0)
        logits = jnp.dot(pooled, wf_ref[...], preferred_element_type=jnp.float32)
        logits = logits + bf_ref[...]
        m = jnp.max(logits, axis=1, keepdims=True)
        lse = jnp.log(jnp.sum(jnp.exp(logits - m), axis=1, keepdims=True)) + m
        o_ref[...] = logits - lse


@jax.jit
def _dense2_pool(eps_s, h1, p0, p1, w, b, batch_r, wf, bf):
    return pl.pallas_call(
        _dense2_body,
        grid=(GRID,),
        in_specs=[
            pl.BlockSpec(memory_space=pltpu.MemorySpace.SMEM),
            pl.BlockSpec((R, H), lambda i: (i, 0)),
            pl.BlockSpec((R, H), lambda i: (i, 0)),
            pl.BlockSpec((R, H), lambda i: (i, 0)),
            pl.BlockSpec((H, H), lambda i: (0, 0)),
            pl.BlockSpec((1, H), lambda i: (0, 0)),
            pl.BlockSpec((1, 1, R), lambda i: (i, 0, 0)),
            pl.BlockSpec((H, 32), lambda i: (0, 0)),
            pl.BlockSpec((1, 32), lambda i: (0, 0)),
        ],
        out_specs=pl.BlockSpec((G, 32), lambda i: (0, 0)),
        out_shape=jax.ShapeDtypeStruct((G, 32), jnp.float32),
        scratch_shapes=[
            pltpu.VMEM((G, H), jnp.float32),
            pltpu.VMEM((G, H), jnp.float32),
        ],
    )(eps_s, h1, p0, p1, w, b, batch_r, wf, bf)


def kernel(x, edge_index, batch, eps1, W1, b1, eps2, W2, b2, Wf, bf):
    npad = EPAD - E
    pad_src = (jnp.arange(npad, dtype=jnp.int32) * 37) % N
    pad_dst = N + (jnp.arange(npad, dtype=jnp.int32) % ND)
    srcs = jnp.concatenate([edge_index[0], pad_src]).reshape(NW, NCH, K)
    dsts = jnp.concatenate([edge_index[1], pad_dst]).reshape(NW, NCH, K)
    zeros = jnp.zeros((RLEN, H), dtype=jnp.float32)
    batch_r = batch.reshape(GRID, 1, R)
    e1 = (1.0 + eps1).reshape(1, 1)
    e2 = (1.0 + eps2).reshape(1, 1)
    b1r = b1.reshape(1, H)
    b2r = b2.reshape(1, H)
    bfr = bf.reshape(1, 32)

    p = _edge_agg(x, srcs, dsts, zeros)
    h1 = _dense1(e1, x, p[0], p[1], W1, b1r)
    p2 = _edge_agg(h1, srcs, dsts, zeros)
    return _dense2_pool(e2, h1, p2[0], p2[1], W2, b2r, batch_r, Wf, bfr)


# speedup vs baseline: 1.1142x; 1.1142x over previous
import functools

import jax
import jax.numpy as jnp
from jax import lax
from jax.experimental import pallas as pl
from jax.experimental.pallas import tpu as pltpu
from jax.experimental.pallas import tpu_sc as plsc

N = 10000
E = 320000
H = 128
G = 64

NC = 2
NS = 16
NW = NC * NS
K = 64
NCH = 160
NST = 4
HC = NCH // NST
NB = 4
EPAD = NW * NCH * K
ND = 16
NA = N + ND
RSTEP = 624
RLEN = 640

R = 1000
GRID = N // R


def _agg_body(h_hbm, srcs_hbm, dsts_hbm, zeros_hbm, out_hbm,
              allis, allid, rows, sems, acc):
    c = lax.axis_index("c")
    s = lax.axis_index("s")
    wid = s * NC + c
    pltpu.sync_copy(zeros_hbm, acc.at[pl.ds(s * RSTEP, RLEN)])
    plsc.subcore_barrier()

    def half(hf, carry):
        pltpu.sync_copy(srcs_hbm.at[wid, pl.ds(hf * HC, HC)], allis)
        pltpu.sync_copy(dsts_hbm.at[wid, pl.ds(hf * HC, HC)], allid)
        for k in range(NB):
            pltpu.async_copy(h_hbm.at[allis.at[k]], rows.at[k], sems.at[k])

        def body(i, carry2):
            base = NB * i
            for k in range(NB):
                j = base + k
                pltpu.make_async_copy(h_hbm.at[allis.at[j]], rows.at[k],
                                      sems.at[k]).wait()
                pltpu.sync_copy(rows.at[k], acc.at[allid.at[j]], add=True)

                @pl.when(j + NB < HC)
                def _():
                    pltpu.async_copy(h_hbm.at[allis.at[j + NB]], rows.at[k],
                                     sems.at[k])
            return carry2

        lax.fori_loop(0, HC // NB, body, 0)
        return carry

    lax.fori_loop(0, NST, half, 0)
    plsc.subcore_barrier()
    pltpu.sync_copy(acc.at[pl.ds(s * RSTEP, RLEN)],
                    out_hbm.at[c, pl.ds(s * RSTEP, RLEN)])


@jax.jit
def _edge_agg(h, srcs, dsts, zeros):
    mesh = plsc.VectorSubcoreMesh(core_axis_name="c", subcore_axis_name="s")
    return pl.kernel(
        _agg_body,
        out_type=jax.ShapeDtypeStruct((NC, N, H), jnp.float32),
        mesh=mesh,
        scratch_types=[
            pltpu.VMEM((HC, K), jnp.int32),
            pltpu.VMEM((HC, K), jnp.int32),
            pltpu.VMEM((NB, K, H), jnp.float32),
            pltpu.SemaphoreType.DMA((NB,)),
            pltpu.VMEM_SHARED((NA, H), jnp.float32),
        ],
    )(h, srcs, dsts, zeros)


def _dense1_body(eps_ref, x_ref, p_ref, w_ref, b_ref, o_ref):
    t = eps_ref[0, 0] * x_ref[...] + p_ref[0] + p_ref[1]
    acc = jnp.dot(t, w_ref[...], preferred_element_type=jnp.float32)
    o_ref[...] = jnp.maximum(acc + b_ref[...], 0.0)


@jax.jit
def _dense1(eps_s, x, p, w, b):
    return pl.pallas_call(
        _dense1_body,
        grid=(GRID,),
        in_specs=[
            pl.BlockSpec(memory_space=pltpu.MemorySpace.SMEM),
            pl.BlockSpec((R, H), lambda i: (i, 0)),
            pl.BlockSpec((NC, R, H), lambda i: (0, i, 0)),
            pl.BlockSpec((H, H), lambda i: (0, 0)),
            pl.BlockSpec((1, H), lambda i: (0, 0)),
        ],
        out_specs=pl.BlockSpec((R, H), lambda i: (i, 0)),
        out_shape=jax.ShapeDtypeStruct((N, H), jnp.float32),
    )(eps_s, x, p, w, b)


def _dense2_body(eps_ref, h_ref, p_ref, w_ref, b_ref, batch_ref,
                 wf_ref, bf_ref, o_ref, sums, counts):
    i = pl.program_id(0)
    t = eps_ref[0, 0] * h_ref[...] + p_ref[0] + p_ref[1]
    h2 = jnp.dot(t, w_ref[...], preferred_element_type=jnp.float32)
    h2 = jnp.maximum(h2 + b_ref[...], 0.0)
    ids = batch_ref[0, 0, :]
    gid = lax.broadcasted_iota(jnp.int32, (G, R), 0)
    mask = (ids[None, :] == gid).astype(jnp.float32)
    psum = jnp.dot(mask, h2, preferred_element_type=jnp.float32)
    pcnt = jnp.broadcast_to(jnp.sum(mask, axis=1, keepdims=True), (G, H))

    @pl.when(i == 0)
    def _():
        sums[...] = psum
        counts[...] = pcnt

    @pl.when(i > 0)
    def _():
        sums[...] += psum
        counts[...] += pcnt

    @pl.when(i == pl.num_programs(0) - 1)
    def _():
        pooled = sums[...] / jnp.maximum(counts[...], 1.0)
        logits = jnp.dot(pooled, wf_ref[...], preferred_element_type=jnp.float32)
        logits = logits + bf_ref[...]
        m = jnp.max(logits, axis=1, keepdims=True)
        lse = jnp.log(jnp.sum(jnp.exp(logits - m), axis=1, keepdims=True)) + m
        o_ref[...] = logits - lse


@jax.jit
def _dense2_pool(eps_s, h1, p, w, b, batch_r, wf, bf):
    return pl.pallas_call(
        _dense2_body,
        grid=(GRID,),
        in_specs=[
            pl.BlockSpec(memory_space=pltpu.MemorySpace.SMEM),
            pl.BlockSpec((R, H), lambda i: (i, 0)),
            pl.BlockSpec((NC, R, H), lambda i: (0, i, 0)),
            pl.BlockSpec((H, H), lambda i: (0, 0)),
            pl.BlockSpec((1, H), lambda i: (0, 0)),
            pl.BlockSpec((1, 1, R), lambda i: (i, 0, 0)),
            pl.BlockSpec((H, 32), lambda i: (0, 0)),
            pl.BlockSpec((1, 32), lambda i: (0, 0)),
        ],
        out_specs=pl.BlockSpec((G, 32), lambda i: (0, 0)),
        out_shape=jax.ShapeDtypeStruct((G, 32), jnp.float32),
        scratch_shapes=[
            pltpu.VMEM((G, H), jnp.float32),
            pltpu.VMEM((G, H), jnp.float32),
        ],
    )(eps_s, h1, p, w, b, batch_r, wf, bf)


def kernel(x, edge_index, batch, eps1, W1, b1, eps2, W2, b2, Wf, bf):
    npad = EPAD - E
    pad_src = (jnp.arange(npad, dtype=jnp.int32) * 37) % N
    pad_dst = N + (jnp.arange(npad, dtype=jnp.int32) % ND)
    srcs = jnp.concatenate([edge_index[0], pad_src]).reshape(NW, NCH, K)
    dsts = jnp.concatenate([edge_index[1], pad_dst]).reshape(NW, NCH, K)
    zeros = jnp.zeros((RLEN, H), dtype=jnp.float32)
    batch_r = batch.reshape(GRID, 1, R)
    e1 = (1.0 + eps1).reshape(1, 1)
    e2 = (1.0 + eps2).reshape(1, 1)
    b1r = b1.reshape(1, H)
    b2r = b2.reshape(1, H)
    bfr = bf.reshape(1, 32)

    p = _edge_agg(x, srcs, dsts, zeros)
    h1 = _dense1(e1, x, p, W1, b1r)
    p2 = _edge_agg(h1, srcs, dsts, zeros)
    return _dense2_pool(e2, h1, p2, W2, b2r, batch_r, Wf, bfr)
